# baseline (speedup 1.0000x reference)
import functools

import jax
import jax.numpy as jnp
from jax import lax
from jax.experimental import pallas as pl
from jax.experimental.pallas import tpu as pltpu
from jax.experimental.pallas import tpu_sc as plsc

N = 10000
E = 160000
D = 256
DH = 128
NC = 2
NS = 16
CH = 128
K = 80
HK = 40
EPAD = NS * CH * K
NPAD = 10240
RPS = NPAD // NS
TRASH = N


def _sc_body(with_cnt, x2, srcs_a, srcs_b, dsts, agg_out, cnt_out,
             agg_sh, cnt_sh, src_v, dst_v, buf, ones_v, cbuf,
             semg0, semg1, sems0, sems1):
    cid = lax.axis_index("c")
    sid = lax.axis_index("s")

    @pl.loop(0, CH)
    def _(i):
        for j in range(DH // 16):
            buf[0, i, pl.ds(j * 16, 16)] = jnp.zeros((16,), jnp.float32)

    @pl.loop(0, CH // 16)
    def _(i):
        ones_v[pl.ds(i * 16, 16)] = jnp.ones((16,), jnp.float32)

    @pl.loop(0, RPS // 16)
    def _(i):
        cbuf[pl.ds(i * 16, 16)] = jnp.zeros((16,), jnp.float32)

    for r in range(RPS // CH):
        pltpu.sync_copy(buf.at[0], agg_sh.at[pl.ds(sid * RPS + r * CH, CH)])
    pltpu.sync_copy(cbuf, cnt_sh.at[pl.ds(sid * RPS, RPS)])
    plsc.subcore_barrier()

    semg = (semg0, semg1)

    def edge_loop(srcs, do_cnt):
        for half in range(K // HK):
            pltpu.sync_copy(srcs.at[sid, pl.ds(half * HK, HK)], src_v)
            pltpu.sync_copy(dsts.at[sid, pl.ds(half * HK, HK)], dst_v)
            pltpu.async_copy(x2.at[src_v.at[0]], buf.at[0], semg[0])

            @pl.loop(0, HK, step=2)
            def _(j):
                for b in range(2):
                    cur = j + b

                    @pl.when(cur + 1 < HK)
                    def _():
                        pltpu.async_copy(x2.at[src_v.at[cur + 1]],
                                         buf.at[1 - b], semg[1 - b])
                    pltpu.make_async_copy(x2.at[pl.ds(0, CH)],
                                          buf.at[b], semg[b]).wait()
                    pltpu.sync_copy(buf.at[b], agg_sh.at[dst_v.at[cur]],
                                    add=True)
                    if do_cnt:
                        pltpu.sync_copy(ones_v, cnt_sh.at[dst_v.at[cur]],
                                        add=True)

    @pl.when(cid == 0)
    def _():
        edge_loop(srcs_a, with_cnt)

    @pl.when(cid == 1)
    def _():
        edge_loop(srcs_b, False)

    plsc.subcore_barrier()

    for r in range(RPS // CH):
        rows = pl.ds(sid * RPS + r * CH, CH)
        pltpu.sync_copy(agg_sh.at[rows], buf.at[0])
        pltpu.sync_copy(buf.at[0], agg_out.at[cid, rows])
    if with_cnt:
        @pl.when(cid == 0)
        def _():
            pltpu.sync_copy(cnt_sh.at[pl.ds(sid * RPS, RPS)], cbuf)
            pltpu.sync_copy(cbuf, cnt_out.at[pl.ds(sid * RPS, RPS)])


def _make_sc_agg(with_cnt):
    mesh = plsc.VectorSubcoreMesh(core_axis_name="c", subcore_axis_name="s")
    out_type = (jax.ShapeDtypeStruct((NC, NPAD, DH), jnp.float32),
                jax.ShapeDtypeStruct((NPAD,), jnp.float32))
    return pl.kernel(
        functools.partial(_sc_body, with_cnt),
        out_type,
        mesh=mesh,
        scratch_types=[
            pltpu.VMEM_SHARED((NPAD, DH), jnp.float32),
            pltpu.VMEM_SHARED((NPAD,), jnp.float32),
            pltpu.VMEM((HK, CH), jnp.int32),
            pltpu.VMEM((HK, CH), jnp.int32),
            pltpu.VMEM((2, CH, DH), jnp.float32),
            pltpu.VMEM((CH,), jnp.float32),
            pltpu.VMEM((RPS,), jnp.float32),
            pltpu.SemaphoreType.DMA,
            pltpu.SemaphoreType.DMA,
            pltpu.SemaphoreType.DMA,
            pltpu.SemaphoreType.DMA,
        ],
        name=f"sage_sc_agg_cnt{int(with_cnt)}",
    )


RB = 400


def _tc_layer_body(relu, agga, aggb, cntr, h, wl, wr, b, out):
    cnt = cntr[...]
    rinv = 1.0 / jnp.maximum(cnt, 1.0)
    mean = jnp.concatenate([agga[0], aggb[0]], axis=1) * rinv
    acc = jnp.dot(mean, wl[...], preferred_element_type=jnp.float32)
    acc += jnp.dot(h[...], wr[...], preferred_element_type=jnp.float32)
    acc += b[...]
    if relu:
        acc = jnp.maximum(acc, 0.0)
    out[...] = acc


def _tc_layer(relu, agg, cntr, h, wl, wr, b):
    grid = N // RB
    wspec = pl.BlockSpec((D, D), lambda i: (0, 0))
    rspec = pl.BlockSpec((RB, D), lambda i: (i, 0))
    return pl.pallas_call(
        functools.partial(_tc_layer_body, relu),
        grid=(grid,),
        in_specs=[
            pl.BlockSpec((1, RB, DH), lambda i: (0, i, 0)),
            pl.BlockSpec((1, RB, DH), lambda i: (1, i, 0)),
            pl.BlockSpec((RB, 1), lambda i: (i, 0)),
            rspec,
            wspec,
            wspec,
            pl.BlockSpec((1, D), lambda i: (0, 0)),
        ],
        out_specs=rspec,
        out_shape=jax.ShapeDtypeStruct((N, D), jnp.float32),
        name=f"sage_tc_layer_relu{int(relu)}",
    )(agg, agg, cntr, h, wl, wr, b)


def kernel(x, edge_index, W1l, W1r, b1, W2l, W2r, b2):
    src = edge_index[0].astype(jnp.int32)
    dst = edge_index[1].astype(jnp.int32)
    srcp = jnp.concatenate([src, jnp.zeros((EPAD - E,), jnp.int32)])
    dstp = jnp.concatenate([dst, jnp.full((EPAD - E,), TRASH, jnp.int32)])
    srcs_a = (2 * srcp).reshape(NS, K, CH)
    srcs_b = (2 * srcp + 1).reshape(NS, K, CH)
    dsts = dstp.reshape(NS, K, CH)

    agg1, cnt1 = _make_sc_agg(True)(x.reshape(2 * N, DH), srcs_a, srcs_b,
                                    dsts)
    cntr = cnt1.reshape(NPAD, 1)

    h = _tc_layer(True, agg1, cntr, x, W1l.T, W1r.T, b1.reshape(1, D))

    agg2, _ = _make_sc_agg(False)(h.reshape(2 * N, DH), srcs_a, srcs_b,
                                  dsts)
    out = _tc_layer(False, agg2, cntr, h, W2l.T, W2r.T, b2.reshape(1, D))
    return out


# speedup vs baseline: 1.0614x; 1.0614x over previous
import functools

import jax
import jax.numpy as jnp
from jax import lax
from jax.experimental import pallas as pl
from jax.experimental.pallas import tpu as pltpu
from jax.experimental.pallas import tpu_sc as plsc

N = 10000
E = 160000
D = 256
DH = 128
NC = 2
NS = 16
CH = 128
K = 80
HK = 40
EPAD = NS * CH * K
NPAD = 10240
RPS = NPAD // NS
TRASH = N


def _sc_body(with_cnt, xa, xb, srcs, dsts, agg_out, cnt_out,
             agg_sh, cnt_sh, src_v, dst_v, buf, ones_v, cbuf,
             semg0, semg1, sems0, sems1):
    cid = lax.axis_index("c")
    sid = lax.axis_index("s")

    @pl.loop(0, CH)
    def _(i):
        for j in range(DH // 16):
            buf[0, i, pl.ds(j * 16, 16)] = jnp.zeros((16,), jnp.float32)

    @pl.loop(0, CH // 16)
    def _(i):
        ones_v[pl.ds(i * 16, 16)] = jnp.ones((16,), jnp.float32)

    @pl.loop(0, RPS // 16)
    def _(i):
        cbuf[pl.ds(i * 16, 16)] = jnp.zeros((16,), jnp.float32)

    for r in range(RPS // CH):
        pltpu.sync_copy(buf.at[0], agg_sh.at[pl.ds(sid * RPS + r * CH, CH)])
    pltpu.sync_copy(cbuf, cnt_sh.at[pl.ds(sid * RPS, RPS)])
    plsc.subcore_barrier()

    semg = (semg0, semg1)

    def edge_loop(x_ref, cnt_halves):
        for half in range(K // HK):
            do_cnt = cnt_halves[half]
            pltpu.sync_copy(srcs.at[sid, pl.ds(half * HK, HK)], src_v)
            pltpu.sync_copy(dsts.at[sid, pl.ds(half * HK, HK)], dst_v)
            pltpu.async_copy(x_ref.at[src_v.at[0]], buf.at[0], semg[0])

            @pl.loop(0, HK, step=2)
            def _(j):
                for b in range(2):
                    cur = j + b

                    @pl.when(cur + 1 < HK)
                    def _():
                        pltpu.async_copy(x_ref.at[src_v.at[cur + 1]],
                                         buf.at[1 - b], semg[1 - b])
                    pltpu.make_async_copy(x_ref.at[src_v.at[cur]],
                                          buf.at[b], semg[b]).wait()
                    pltpu.sync_copy(buf.at[b], agg_sh.at[dst_v.at[cur]],
                                    add=True)
                    if do_cnt:
                        pltpu.sync_copy(ones_v, cnt_sh.at[dst_v.at[cur]],
                                        add=True)

    @pl.when(cid == 0)
    def _():
        edge_loop(xa, (with_cnt, with_cnt))

    @pl.when(cid == 1)
    def _():
        edge_loop(xb, (False, False))

    plsc.subcore_barrier()

    for r in range(RPS // CH):
        rows = pl.ds(sid * RPS + r * CH, CH)
        pltpu.sync_copy(agg_sh.at[rows], buf.at[0])
        pltpu.sync_copy(buf.at[0], agg_out.at[cid, rows])
    if with_cnt:
        @pl.when(cid == 0)
        def _():
            pltpu.sync_copy(cnt_sh.at[pl.ds(sid * RPS, RPS)], cbuf)
            pltpu.sync_copy(cbuf, cnt_out.at[pl.ds(sid * RPS, RPS)])


def _make_sc_agg(with_cnt):
    mesh = plsc.VectorSubcoreMesh(core_axis_name="c", subcore_axis_name="s")
    out_type = (jax.ShapeDtypeStruct((NC, NPAD, DH), jnp.float32),
                jax.ShapeDtypeStruct((NPAD,), jnp.float32))
    return pl.kernel(
        functools.partial(_sc_body, with_cnt),
        out_type,
        mesh=mesh,
        scratch_types=[
            pltpu.VMEM_SHARED((NPAD, DH), jnp.float32),
            pltpu.VMEM_SHARED((NPAD,), jnp.float32),
            pltpu.VMEM((HK, CH), jnp.int32),
            pltpu.VMEM((HK, CH), jnp.int32),
            pltpu.VMEM((2, CH, DH), jnp.float32),
            pltpu.VMEM((CH,), jnp.float32),
            pltpu.VMEM((RPS,), jnp.float32),
            pltpu.SemaphoreType.DMA,
            pltpu.SemaphoreType.DMA,
            pltpu.SemaphoreType.DMA,
            pltpu.SemaphoreType.DMA,
        ],
        name=f"sage_sc_agg_cnt{int(with_cnt)}",
    )


RB = 400


def _tc_layer_body(split_out, relu, agga, aggb, cnta, cntb, ha, hb,
                   wl, wra, wrb, b, *outs):
    cnt = cnta[...] + cntb[...]
    rinv = 1.0 / jnp.maximum(cnt, 1.0)
    mean = jnp.concatenate([agga[0], aggb[0]], axis=1) * rinv
    acc = jnp.dot(mean, wl[...], preferred_element_type=jnp.float32)
    acc += jnp.dot(ha[...], wra[...], preferred_element_type=jnp.float32)
    acc += jnp.dot(hb[...], wrb[...], preferred_element_type=jnp.float32)
    acc += b[...]
    if relu:
        acc = jnp.maximum(acc, 0.0)
    if split_out:
        outs[0][...] = acc[:, :DH]
        outs[1][...] = acc[:, DH:]
    else:
        outs[0][...] = acc


def _tc_layer(split_out, relu, agg, cnta, cntb, ha, hb, wl, wra, wrb, b):
    grid = N // RB
    wspec = pl.BlockSpec((D, D), lambda i: (0, 0))
    hspec = pl.BlockSpec((RB, DH), lambda i: (i, 0))
    cspec = pl.BlockSpec((RB, 1), lambda i: (i, 0))
    if split_out:
        out_specs = (hspec, hspec)
        out_shape = (jax.ShapeDtypeStruct((N, DH), jnp.float32),
                     jax.ShapeDtypeStruct((N, DH), jnp.float32))
    else:
        out_specs = pl.BlockSpec((RB, D), lambda i: (i, 0))
        out_shape = jax.ShapeDtypeStruct((N, D), jnp.float32)
    return pl.pallas_call(
        functools.partial(_tc_layer_body, split_out, relu),
        grid=(grid,),
        in_specs=[
            pl.BlockSpec((1, RB, DH), lambda i: (0, i, 0)),
            pl.BlockSpec((1, RB, DH), lambda i: (1, i, 0)),
            cspec,
            cspec,
            hspec,
            hspec,
            wspec,
            pl.BlockSpec((DH, D), lambda i: (0, 0)),
            pl.BlockSpec((DH, D), lambda i: (0, 0)),
            pl.BlockSpec((1, D), lambda i: (0, 0)),
        ],
        out_specs=out_specs,
        out_shape=out_shape,
        name=f"sage_tc_layer_relu{int(relu)}",
    )(agg, agg, cnta, cntb, ha, hb, wl, wra, wrb, b)


def kernel(x, edge_index, W1l, W1r, b1, W2l, W2r, b2):
    src = edge_index[0].astype(jnp.int32)
    dst = edge_index[1].astype(jnp.int32)
    srcp = jnp.concatenate([src, jnp.zeros((EPAD - E,), jnp.int32)])
    dstp = jnp.concatenate([dst, jnp.full((EPAD - E,), TRASH, jnp.int32)])
    srcs = srcp.reshape(NS, K, CH)
    dsts = dstp.reshape(NS, K, CH)

    xa = x[:, :DH]
    xb = x[:, DH:]

    agg1, cnt1 = _make_sc_agg(True)(xa, xb, srcs, dsts)
    cnta = cnt1.reshape(NPAD, 1)
    cntb = jnp.zeros((NPAD, 1), jnp.float32)

    w1l = W1l.T
    w1ra = W1r.T[:DH]
    w1rb = W1r.T[DH:]
    ha, hb = _tc_layer(True, True, agg1, cnta, cntb, xa, xb,
                       w1l, w1ra, w1rb, b1.reshape(1, D))

    agg2, _ = _make_sc_agg(False)(ha, hb, srcs, dsts)
    out = _tc_layer(False, False, agg2, cnta, cntb, ha, hb,
                    W2l.T, W2r.T[:DH], W2r.T[DH:], b2.reshape(1, D))
    return out
